# Initial kernel scaffold; baseline (speedup 1.0000x reference)
import jax
import jax.numpy as jnp
from jax.experimental import pallas as pl


def kernel(x, edge_index, W1, b1, W2, b2):
    raise NotImplementedError("write your pallas kernel here")


# speedup vs baseline: 12.5629x; 12.5629x over previous
import functools

import jax
import jax.numpy as jnp
from jax import lax
from jax.experimental import pallas as pl
from jax.experimental.pallas import tpu as pltpu
import jax.experimental.pallas.tpu_sc as plsc

N = 10000
E = 320000
D_IN = 128
H = 256
HH = H // 2

NC = 2
NT = 16
CH = 128
KCH = 157
EPAD = NT * KCH * CH
ACC_ROWS = 10112

_f32 = jnp.float32
_i32 = jnp.int32


def _deg_body(dst_hbm, deg_hbm, dst_v, ones_v, buf_v, deg_sp, sem):
    c = lax.axis_index("c")
    s = lax.axis_index("s")

    def fill_ones(i, _):
        ones_v[pl.ds(i * 16, 16)] = jnp.ones((16,), _f32)
        return 0

    lax.fori_loop(0, 64, fill_ones, 0)

    on_core0 = c == 0

    @pl.when(jnp.logical_and(on_core0, s < 10))
    def _():
        pltpu.sync_copy(ones_v.at[pl.ds(0, 1000)], deg_sp.at[pl.ds(s * 1000, 1000)])

    plsc.subcore_barrier()

    @pl.when(on_core0)
    def _():
        for blk, cnt in [(0, 32), (1, 32), (2, 32), (3, 32), (4, 29)]:
            pltpu.sync_copy(dst_hbm.at[s, pl.ds(blk * 32, cnt)], dst_v.at[pl.ds(0, cnt)])

            def step(j, _):
                pltpu.sync_copy(ones_v.at[pl.ds(0, CH)], deg_sp.at[dst_v.at[j]], add=True)
                return 0

            lax.fori_loop(0, cnt, step, 0)

    plsc.subcore_barrier()

    @pl.when(jnp.logical_and(on_core0, s < 10))
    def _():
        pltpu.sync_copy(deg_sp.at[pl.ds(s * 1000, 1000)], buf_v)
        pltpu.sync_copy(buf_v, deg_hbm.at[pl.ds(s * 1000, 1000)])


def _sc_degree(dst_t):
    mesh = plsc.VectorSubcoreMesh(core_axis_name="c", subcore_axis_name="s")
    return pl.kernel(
        _deg_body,
        out_type=jax.ShapeDtypeStruct((N,), _f32),
        mesh=mesh,
        scratch_types=[
            pltpu.VMEM((32, CH), _i32),
            pltpu.VMEM((1024,), _f32),
            pltpu.VMEM((1000,), _f32),
            pltpu.VMEM_SHARED((ACC_ROWS,), _f32),
            pltpu.SemaphoreType.DMA,
        ],
    )(dst_t)


def _agg_body(y_hbm, src_hbm, dst_hbm, out_hbm, src_v, dst_v, buf0, acc_sp, sem):
    c = lax.axis_index("c")
    s = lax.axis_index("s")

    def fill_zero(i, _):
        buf0[i // 8, pl.ds((i % 8) * 16, 16)] = jnp.zeros((16,), _f32)
        return 0

    lax.fori_loop(0, 1024, fill_zero, 0)

    base = s * 632
    for q in range(4):
        pltpu.sync_copy(buf0, acc_sp.at[pl.ds(base + q * CH, CH)])
    pltpu.sync_copy(buf0.at[pl.ds(0, 120)], acc_sp.at[pl.ds(base + 512, 120)])

    plsc.subcore_barrier()

    for blk, cnt in [(0, 32), (1, 32), (2, 32), (3, 32), (4, 29)]:
        pltpu.sync_copy(src_hbm.at[c, s, pl.ds(blk * 32, cnt)], src_v.at[pl.ds(0, cnt)])
        pltpu.sync_copy(dst_hbm.at[s, pl.ds(blk * 32, cnt)], dst_v.at[pl.ds(0, cnt)])

        def step(j, _):
            pltpu.async_copy(y_hbm.at[src_v.at[j]], buf0, sem).wait()
            pltpu.sync_copy(buf0, acc_sp.at[dst_v.at[j]], add=True)
            return 0

        lax.fori_loop(0, cnt, step, 0)

    plsc.subcore_barrier()

    for q, (off, sz) in enumerate([(0, CH), (CH, CH), (2 * CH, CH), (3 * CH, CH), (4 * CH, 112)]):
        row = s * 624 + off
        pltpu.sync_copy(acc_sp.at[pl.ds(row, sz)], buf0.at[pl.ds(0, sz)])
        pltpu.sync_copy(buf0.at[pl.ds(0, sz)], out_hbm.at[pl.ds(c * N + row, sz)])

    @pl.when(s == 0)
    def _():
        pltpu.sync_copy(acc_sp.at[pl.ds(9984, 16)], buf0.at[pl.ds(0, 16)])
        pltpu.sync_copy(buf0.at[pl.ds(0, 16)], out_hbm.at[pl.ds(c * N + 9984, 16)])


def _sc_aggregate(y_cat, src2, dst_t):
    mesh = plsc.VectorSubcoreMesh(core_axis_name="c", subcore_axis_name="s")
    return pl.kernel(
        _agg_body,
        out_type=jax.ShapeDtypeStruct((2 * N, HH), _f32),
        mesh=mesh,
        scratch_types=[
            pltpu.VMEM((32, CH), _i32),
            pltpu.VMEM((32, CH), _i32),
            pltpu.VMEM((CH, HH), _f32),
            pltpu.VMEM_SHARED((ACC_ROWS, HH), _f32),
            pltpu.SemaphoreType.DMA,
        ],
    )(y_cat, src2, dst_t)


_RB = 1000
_NRB = N // _RB

_DOT = functools.partial(
    jnp.dot, preferred_element_type=_f32, precision=jax.lax.Precision.HIGHEST
)


def _tc1_body(x_ref, w_ref, deg_ref, out_ref):
    dinv = lax.rsqrt(deg_ref[...][:, 0])
    xw = _DOT(x_ref[...], w_ref[...])
    out_ref[...] = xw * dinv[:, None]


def _tc1(x, W1, deg):
    return pl.pallas_call(
        _tc1_body,
        grid=(_NRB, 2),
        in_specs=[
            pl.BlockSpec((_RB, D_IN), lambda r, h: (r, 0)),
            pl.BlockSpec((D_IN, HH), lambda r, h: (0, h)),
            pl.BlockSpec((_RB, 1), lambda r, h: (r, 0)),
        ],
        out_specs=pl.BlockSpec((_RB, HH), lambda r, h: (h * _NRB + r, 0)),
        out_shape=jax.ShapeDtypeStruct((2 * N, HH), _f32),
    )(x, W1, deg)


def _tc2_body(alo_ref, ahi_ref, ylo_ref, yhi_ref, deg_ref, b_ref, w_ref, out_ref):
    dinv = lax.rsqrt(deg_ref[...])
    h_lo = jax.nn.relu(dinv * (alo_ref[...] + ylo_ref[...]) + b_ref[0:1, :HH])
    h_hi = jax.nn.relu(dinv * (ahi_ref[...] + yhi_ref[...]) + b_ref[0:1, HH:])
    hcat = jnp.concatenate([h_lo, h_hi], axis=1)
    out_ref[...] = _DOT(hcat, w_ref[...]) * dinv


def _tc2(agg1, y1, deg, b1, W2):
    b2d = b1[None, :]
    return pl.pallas_call(
        _tc2_body,
        grid=(_NRB, 2),
        in_specs=[
            pl.BlockSpec((_RB, HH), lambda r, h: (r, 0)),
            pl.BlockSpec((_RB, HH), lambda r, h: (_NRB + r, 0)),
            pl.BlockSpec((_RB, HH), lambda r, h: (r, 0)),
            pl.BlockSpec((_RB, HH), lambda r, h: (_NRB + r, 0)),
            pl.BlockSpec((_RB, 1), lambda r, h: (r, 0)),
            pl.BlockSpec((1, H), lambda r, h: (0, 0)),
            pl.BlockSpec((H, HH), lambda r, h: (0, h)),
        ],
        out_specs=pl.BlockSpec((_RB, HH), lambda r, h: (h * _NRB + r, 0)),
        out_shape=jax.ShapeDtypeStruct((2 * N, HH), _f32),
    )(agg1, agg1, y1, y1, deg, b2d, W2)


def _tc3_body(alo_ref, ahi_ref, ylo_ref, yhi_ref, deg_ref, b_ref, out_ref):
    dinv = lax.rsqrt(deg_ref[...])
    h_lo = jax.nn.relu(dinv * (alo_ref[...] + ylo_ref[...]) + b_ref[0:1, :HH])
    h_hi = jax.nn.relu(dinv * (ahi_ref[...] + yhi_ref[...]) + b_ref[0:1, HH:])
    out_ref[...] = jnp.concatenate([h_lo, h_hi], axis=1)


def _tc3(agg2, y2, deg, b2):
    b2d = b2[None, :]
    return pl.pallas_call(
        _tc3_body,
        grid=(_NRB,),
        in_specs=[
            pl.BlockSpec((_RB, HH), lambda r: (r, 0)),
            pl.BlockSpec((_RB, HH), lambda r: (_NRB + r, 0)),
            pl.BlockSpec((_RB, HH), lambda r: (r, 0)),
            pl.BlockSpec((_RB, HH), lambda r: (_NRB + r, 0)),
            pl.BlockSpec((_RB, 1), lambda r: (r, 0)),
            pl.BlockSpec((1, H), lambda r: (0, 0)),
        ],
        out_specs=pl.BlockSpec((_RB, H), lambda r: (r, 0)),
        out_shape=jax.ShapeDtypeStruct((N, H), _f32),
    )(agg2, agg2, y2, y2, deg, b2d)


def kernel(x, edge_index, W1, b1, W2, b2):
    src = edge_index[0].astype(_i32)
    dst = edge_index[1].astype(_i32)

    pad = EPAD - E
    src_p = jnp.concatenate([src, jnp.zeros((pad,), _i32)])
    dst_p = jnp.concatenate([dst, jnp.full((pad,), N, _i32)])
    src_t = src_p.reshape(NT, KCH, CH)
    dst_t = dst_p.reshape(NT, KCH, CH)
    src2 = jnp.stack([src_t, src_t + N])

    deg = _sc_degree(dst_t)[:, None]
    y1 = _tc1(x, W1, deg)
    agg1 = _sc_aggregate(y1, src2, dst_t)
    y2 = _tc2(agg1, y1, deg, b1, W2)
    agg2 = _sc_aggregate(y2, src2, dst_t)
    out = _tc3(agg2, y2, deg, b2)
    return out
